# baseline (device time: 29120 ns/iter reference)
import jax
import jax.numpy as jnp
from jax import lax
from jax.experimental import pallas as pl
from jax.experimental.pallas import tpu as pltpu

N_DEV = 8
N_LAYERS = 3
PHASE2_MASKS = (3, 4, 7)


def kernel(x, Win0, Wout0, Win1, Wout1, Win2, Wout2):
    b, d_sh = x.shape
    hdim = Win0.shape[1]

    def body(x_ref, win0_ref, wout0_ref, win1_ref, wout1_ref, win2_ref,
             wout2_ref, out_ref, send_buf, recv1_ref, pair_buf, recv2_ref,
             s1_sems, r1_sems, s2_sems, r2_sems):
        my = lax.axis_index("i")
        wins = [win0_ref, win1_ref, win2_ref]
        wouts = [wout0_ref, wout1_ref, wout2_ref]

        barrier_sem = pltpu.get_barrier_semaphore()
        pl.semaphore_signal(barrier_sem, inc=1, device_id=(my,),
                            device_id_type=pl.DeviceIdType.MESH)
        pl.semaphore_wait(barrier_sem, 1)

        all_rdmas = []
        x_cur = x_ref[...].astype(jnp.bfloat16)
        for l in range(N_LAYERS):
            partial = jnp.dot(
                x_cur, wins[l][...].astype(jnp.bfloat16),
                preferred_element_type=jnp.float32,
            ).astype(jnp.bfloat16)
            send_buf[l] = partial

            rdma1 = pltpu.make_async_remote_copy(
                src_ref=send_buf.at[l],
                dst_ref=recv1_ref.at[l],
                send_sem=s1_sems.at[l],
                recv_sem=r1_sems.at[l],
                device_id=(my ^ 1,),
                device_id_type=pl.DeviceIdType.MESH,
            )
            rdma1.start()
            all_rdmas.append(rdma1)
            rdma1.wait_recv()
            pair = partial + recv1_ref[l]
            pair_buf[l] = pair

            rdmas2 = []
            for j, m in enumerate(PHASE2_MASKS):
                rdma = pltpu.make_async_remote_copy(
                    src_ref=pair_buf.at[l],
                    dst_ref=recv2_ref.at[l, j],
                    send_sem=s2_sems.at[l, j],
                    recv_sem=r2_sems.at[l, j],
                    device_id=(my ^ m,),
                    device_id_type=pl.DeviceIdType.MESH,
                )
                rdma.start()
                rdmas2.append(rdma)
            all_rdmas.extend(rdmas2)

            acc = pair
            for j in range(len(PHASE2_MASKS)):
                rdmas2[j].wait_recv()
                acc = acc + recv2_ref[l, j]
            h = jnp.maximum(acc, 0)
            x_cur = jnp.dot(
                h, wouts[l][...].astype(jnp.bfloat16),
                preferred_element_type=jnp.float32,
            ).astype(jnp.bfloat16)

        out_ref[...] = x_cur.astype(jnp.float32)
        for rdma in all_rdmas:
            rdma.wait_send()

    return pl.pallas_call(
        body,
        out_shape=jax.ShapeDtypeStruct((b, d_sh), jnp.float32),
        in_specs=[pl.BlockSpec(memory_space=pltpu.VMEM)] * 7,
        out_specs=pl.BlockSpec(memory_space=pltpu.VMEM),
        scratch_shapes=[
            pltpu.VMEM((N_LAYERS, b, hdim), jnp.bfloat16),
            pltpu.VMEM((N_LAYERS, b, hdim), jnp.bfloat16),
            pltpu.VMEM((N_LAYERS, b, hdim), jnp.bfloat16),
            pltpu.VMEM((N_LAYERS, 3, b, hdim), jnp.bfloat16),
            pltpu.SemaphoreType.DMA((N_LAYERS,)),
            pltpu.SemaphoreType.DMA((N_LAYERS,)),
            pltpu.SemaphoreType.DMA((N_LAYERS, 3)),
            pltpu.SemaphoreType.DMA((N_LAYERS, 3)),
        ],
        compiler_params=pltpu.CompilerParams(collective_id=0),
    )(x, Win0, Wout0, Win1, Wout1, Win2, Wout2)


# device time: 26969 ns/iter; 1.0798x vs baseline; 1.0798x over previous
import jax
import jax.numpy as jnp
from jax import lax
from jax.experimental import pallas as pl
from jax.experimental.pallas import tpu as pltpu

N_DEV = 8
N_LAYERS = 3


def kernel(x, Win0, Wout0, Win1, Wout1, Win2, Wout2):
    b, d_sh = x.shape
    hdim = Win0.shape[1]

    def body(x_ref, win0_ref, wout0_ref, win1_ref, wout1_ref, win2_ref,
             wout2_ref, out_ref, send_buf, comm_ref, send_sems, recv_sems):
        my = lax.axis_index("i")
        wins = [win0_ref, win1_ref, win2_ref]
        wouts = [wout0_ref, wout1_ref, wout2_ref]

        barrier_sem = pltpu.get_barrier_semaphore()
        pl.semaphore_signal(barrier_sem, inc=1, device_id=(my,),
                            device_id_type=pl.DeviceIdType.MESH)
        pl.semaphore_wait(barrier_sem, 1)

        all_rdmas = []
        x_cur = x_ref[...].astype(jnp.bfloat16)
        for l in range(N_LAYERS):
            partial = jnp.dot(
                x_cur, wins[l][...].astype(jnp.bfloat16),
                preferred_element_type=jnp.float32,
            ).astype(jnp.bfloat16)
            send_buf[l] = partial

            rdmas = []
            for k in range(1, N_DEV):
                peer = lax.rem(my + k, N_DEV)
                rdma = pltpu.make_async_remote_copy(
                    src_ref=send_buf.at[l],
                    dst_ref=comm_ref.at[l, k - 1],
                    send_sem=send_sems.at[l, k - 1],
                    recv_sem=recv_sems.at[l, k - 1],
                    device_id=(peer,),
                    device_id_type=pl.DeviceIdType.MESH,
                )
                rdma.start()
                rdmas.append(rdma)
            all_rdmas.extend(rdmas)

            acc = partial
            for k in range(1, N_DEV):
                rdmas[k - 1].wait_recv()
                acc = acc + comm_ref[l, k - 1]
            h = jnp.maximum(acc, 0)
            x_cur = jnp.dot(
                h, wouts[l][...].astype(jnp.bfloat16),
                preferred_element_type=jnp.float32,
            ).astype(jnp.bfloat16)

        out_ref[...] = x_cur.astype(jnp.float32)
        for rdma in all_rdmas:
            rdma.wait_send()

    return pl.pallas_call(
        body,
        out_shape=jax.ShapeDtypeStruct((b, d_sh), jnp.float32),
        in_specs=[pl.BlockSpec(memory_space=pltpu.VMEM)] * 7,
        out_specs=pl.BlockSpec(memory_space=pltpu.VMEM),
        scratch_shapes=[
            pltpu.VMEM((N_LAYERS, b, hdim), jnp.bfloat16),
            pltpu.VMEM((N_LAYERS, N_DEV - 1, b, hdim), jnp.bfloat16),
            pltpu.SemaphoreType.DMA((N_LAYERS, N_DEV - 1)),
            pltpu.SemaphoreType.DMA((N_LAYERS, N_DEV - 1)),
        ],
        compiler_params=pltpu.CompilerParams(collective_id=0),
    )(x, Win0, Wout0, Win1, Wout1, Win2, Wout2)


# device time: 26964 ns/iter; 1.0800x vs baseline; 1.0002x over previous
import jax
import jax.numpy as jnp
from jax import lax
from jax.experimental import pallas as pl
from jax.experimental.pallas import tpu as pltpu

N_DEV = 8
N_LAYERS = 3


def kernel(x, Win0, Wout0, Win1, Wout1, Win2, Wout2):
    b, d_sh = x.shape
    hdim = Win0.shape[1]

    def body(x_ref, win0_ref, wout0_ref, win1_ref, wout1_ref, win2_ref,
             wout2_ref, out_ref, send_buf, comm_ref, send_sems, recv_sems):
        my = lax.axis_index("i")
        wins = [win0_ref, win1_ref, win2_ref]
        wouts = [wout0_ref, wout1_ref, wout2_ref]

        barrier_sem = pltpu.get_barrier_semaphore()
        pl.semaphore_signal(barrier_sem, inc=1, device_id=(my,),
                            device_id_type=pl.DeviceIdType.MESH)
        pl.semaphore_wait(barrier_sem, 1)

        all_rdmas = []
        x_cur = x_ref[...].astype(jnp.bfloat16)
        win_bf = win0_ref[...].astype(jnp.bfloat16)
        for l in range(N_LAYERS):
            partial = jnp.dot(
                x_cur, win_bf, preferred_element_type=jnp.float32,
            ).astype(jnp.bfloat16)
            send_buf[l] = partial

            rdmas = []
            for k in range(1, N_DEV):
                peer = lax.rem(my + k, N_DEV)
                rdma = pltpu.make_async_remote_copy(
                    src_ref=send_buf.at[l],
                    dst_ref=comm_ref.at[l, k - 1],
                    send_sem=send_sems.at[l, k - 1],
                    recv_sem=recv_sems.at[l, k - 1],
                    device_id=(peer,),
                    device_id_type=pl.DeviceIdType.MESH,
                )
                rdma.start()
                rdmas.append(rdma)
            all_rdmas.extend(rdmas)

            wout_bf = wouts[l][...].astype(jnp.bfloat16)
            if l + 1 < N_LAYERS:
                win_bf = wins[l + 1][...].astype(jnp.bfloat16)

            acc = partial
            for k in range(1, N_DEV):
                rdmas[k - 1].wait_recv()
                acc = acc + comm_ref[l, k - 1]
            h = jnp.maximum(acc, 0)
            x_cur = jnp.dot(
                h, wout_bf, preferred_element_type=jnp.float32,
            ).astype(jnp.bfloat16)

        out_ref[...] = x_cur.astype(jnp.float32)
        for rdma in all_rdmas:
            rdma.wait_send()

    return pl.pallas_call(
        body,
        out_shape=jax.ShapeDtypeStruct((b, d_sh), jnp.float32),
        in_specs=[pl.BlockSpec(memory_space=pltpu.VMEM)] * 7,
        out_specs=pl.BlockSpec(memory_space=pltpu.VMEM),
        scratch_shapes=[
            pltpu.VMEM((N_LAYERS, b, hdim), jnp.bfloat16),
            pltpu.VMEM((N_LAYERS, N_DEV - 1, b, hdim), jnp.bfloat16),
            pltpu.SemaphoreType.DMA((N_LAYERS, N_DEV - 1)),
            pltpu.SemaphoreType.DMA((N_LAYERS, N_DEV - 1)),
        ],
        compiler_params=pltpu.CompilerParams(collective_id=0),
    )(x, Win0, Wout0, Win1, Wout1, Win2, Wout2)
